# Initial kernel scaffold; baseline (speedup 1.0000x reference)
import jax
import jax.numpy as jnp
from jax.experimental import pallas as pl


def kernel(user_ids, log_seqs, item_ids, params):
    raise NotImplementedError("write your pallas kernel here")


# speedup vs baseline: 1.0972x; 1.0972x over previous
import functools

import jax
import jax.numpy as jnp
from jax import lax
from jax.experimental import pallas as pl
from jax.experimental.pallas import tpu as pltpu
from jax.experimental.pallas import tpu_sc as plsc

HIDDEN = 64
NUM_HEADS = 2
NUM_BLOCKS = 2
NUM_EXPERTS = 4
TOP_K = 2
ALPHA = 0.5

BB = 8
NW = 32
SC_CHUNK = 800


def _sc_gather(item_emb, user_emb, seq_idx, user_ids, item_ids):
    n_seq = seq_idx.shape[0]
    batch = user_ids.shape[0]
    rows_per_w = n_seq // NW
    n_chunks = rows_per_w // SC_CHUNK
    b_per_w = batch // NW

    def body(item_hbm, user_hbm, sidx_hbm, uid_hbm, iid_hbm,
             seq_out, u_out, i_out, idx_v, rows_v, idx_s, rows_s, sem):
        wid = lax.axis_index("s") * 2 + lax.axis_index("c")
        base0 = wid * rows_per_w
        for c in range(n_chunks):
            b = base0 + c * SC_CHUNK
            pltpu.sync_copy(sidx_hbm.at[pl.ds(b, SC_CHUNK)], idx_v)
            pltpu.async_copy(item_hbm.at[idx_v], rows_v, sem).wait()
            pltpu.sync_copy(rows_v, seq_out.at[pl.ds(b, SC_CHUNK)])
        ub = wid * b_per_w
        pltpu.sync_copy(uid_hbm.at[pl.ds(ub, b_per_w)], idx_s)
        pltpu.async_copy(user_hbm.at[idx_s], rows_s, sem).wait()
        pltpu.sync_copy(rows_s, u_out.at[pl.ds(ub, b_per_w)])
        pltpu.sync_copy(iid_hbm.at[pl.ds(ub, b_per_w)], idx_s)
        pltpu.async_copy(item_hbm.at[idx_s], rows_s, sem).wait()
        pltpu.sync_copy(rows_s, i_out.at[pl.ds(ub, b_per_w)])

    kfn = pl.kernel(
        body,
        mesh=plsc.VectorSubcoreMesh(core_axis_name="c", subcore_axis_name="s"),
        out_type=[
            jax.ShapeDtypeStruct((n_seq, HIDDEN), jnp.float32),
            jax.ShapeDtypeStruct((batch, HIDDEN), jnp.float32),
            jax.ShapeDtypeStruct((batch, HIDDEN), jnp.float32),
        ],
        scratch_types=[
            pltpu.VMEM((SC_CHUNK,), jnp.int32),
            pltpu.VMEM((SC_CHUNK, HIDDEN), jnp.float32),
            pltpu.VMEM((b_per_w,), jnp.int32),
            pltpu.VMEM((b_per_w, HIDDEN), jnp.float32),
            pltpu.SemaphoreType.DMA,
        ],
        compiler_params=pltpu.CompilerParams(use_tc_tiling_on_sc=False),
    )
    return kfn(item_emb, user_emb, seq_idx, user_ids, item_ids)


def _flatten_params(params):
    out = []

    def add(a):
        out.append(a if a.ndim == 2 else a.reshape(1, -1))

    add(params["pos_emb"])
    for bp in params["blocks"]:
        add(bp["attn_ln_g"]); add(bp["attn_ln_b"])
        m = bp["mha"]
        add(m["qkv_w"]); add(m["qkv_b"]); add(m["out_w"]); add(m["out_b"])
        add(bp["fwd_ln_g"]); add(bp["fwd_ln_b"])
        g = bp["moe"]["gate"]
        add(g["qkv_w"]); add(g["qkv_b"]); add(g["out_w"]); add(g["out_b"])
        add(g["lin1_w"]); add(g["lin1_b"]); add(g["lin2_w"]); add(g["lin2_b"])
        add(g["ln1_g"]); add(g["ln1_b"]); add(g["ln2_g"]); add(g["ln2_b"])
        add(g["proj_w"]); add(g["proj_b"])
        for ep in bp["moe"]["experts"]:
            add(ep["w1"]); add(ep["b1"]); add(ep["w2"]); add(ep["b2"])
        add(bp["moe"]["ln_g"]); add(bp["moe"]["ln_b"])
    add(params["last_ln_g"]); add(params["last_ln_b"])
    return out


def _ln(x, g, b, eps):
    mu = jnp.mean(x, axis=-1, keepdims=True)
    var = jnp.mean((x - mu) ** 2, axis=-1, keepdims=True)
    return (x - mu) / jnp.sqrt(var + eps) * g + b


def _mha(q_in, kv_in, qkv_w, qkv_b, out_w, out_b, cmask):
    dh = HIDDEN // NUM_HEADS
    w_q, w_k, w_v = qkv_w[0:64], qkv_w[64:128], qkv_w[128:192]
    b_q, b_k, b_v = qkv_b[:, 0:64], qkv_b[:, 64:128], qkv_b[:, 128:192]
    dn = (((2,), (1,)), ((), ()))
    q = lax.dot_general(q_in, w_q, dn, preferred_element_type=jnp.float32) + b_q
    k = lax.dot_general(kv_in, w_k, dn, preferred_element_type=jnp.float32) + b_k
    v = lax.dot_general(kv_in, w_v, dn, preferred_element_type=jnp.float32) + b_v
    outs = []
    for h in range(NUM_HEADS):
        sl = slice(h * dh, (h + 1) * dh)
        qh, kh, vh = q[..., sl], k[..., sl], v[..., sl]
        s = lax.dot_general(qh, kh, (((2,), (2,)), ((0,), (0,))),
                            preferred_element_type=jnp.float32)
        s = s / jnp.sqrt(jnp.float32(dh))
        if cmask is not None:
            s = jnp.where(cmask[None], jnp.float32(-1e9), s)
        a = jax.nn.softmax(s, axis=-1)
        outs.append(lax.dot_general(a, vh, (((2,), (1,)), ((0,), (0,))),
                                    preferred_element_type=jnp.float32))
    o = jnp.concatenate(outs, axis=-1)
    return lax.dot_general(o, out_w, dn, preferred_element_type=jnp.float32) + out_b


def _tc_forward(x_rows, poss, noise, u_rows, i_rows, params):
    B, L, H = x_rows.shape
    grid = B // BB
    wlist = _flatten_params(params)
    nw = len(wlist)

    def body(*refs):
        xr, pr, nr, ur, ir = refs[:5]
        w = [r[...] for r in refs[5:5 + nw]]
        out_l, out_m = refs[5 + nw], refs[6 + nw]
        pop = functools.partial(w.pop, 0)
        step = pl.program_id(0)

        dn = (((2,), (1,)), ((), ()))
        pos_emb = pop()

        x = xr[...] * jnp.float32(8.0)
        pidx = pr[...]
        oh = (pidx[:, :, None]
              == lax.broadcasted_iota(jnp.int32, (1, 1, L + 1), 2)
              ).astype(jnp.float32)
        x = x + lax.dot_general(oh, pos_emb, (((2,), (0,)), ((), ())),
                                preferred_element_type=jnp.float32)

        row = lax.broadcasted_iota(jnp.int32, (L, L), 0)
        col = lax.broadcasted_iota(jnp.int32, (L, L), 1)
        cmask = col > row
        e_iota = lax.broadcasted_iota(jnp.int32, (BB, L, NUM_EXPERTS), 2)

        kl_gw = None
        for bi in range(NUM_BLOCKS):
            attn_g, attn_b = pop(), pop()
            qkv_w, qkv_b, ow, ob = pop(), pop(), pop(), pop()
            fwd_g, fwd_b = pop(), pop()
            g_qkv_w, g_qkv_b, g_ow, g_ob = pop(), pop(), pop(), pop()
            g_l1w, g_l1b, g_l2w, g_l2b = pop(), pop(), pop(), pop()
            g_ln1g, g_ln1b, g_ln2g, g_ln2b = pop(), pop(), pop(), pop()
            g_pw, g_pb = pop(), pop()
            experts = [(pop(), pop(), pop(), pop()) for _ in range(NUM_EXPERTS)]
            moe_g, moe_b = pop(), pop()

            Q = _ln(x, attn_g, attn_b, 1e-8)
            x = Q + _mha(Q, x, qkv_w, qkv_b, ow, ob, cmask)
            x = _ln(x, fwd_g, fwd_b, 1e-8)

            residual = x
            boost = x
            gw = None
            for i in range(TOP_K):
                sa = _mha(boost, boost, g_qkv_w, g_qkv_b, g_ow, g_ob, None)
                t = _ln(boost + sa, g_ln1g, g_ln1b, 1e-5)
                ff = lax.dot_general(
                    jnp.maximum(
                        lax.dot_general(t, g_l1w, dn,
                                        preferred_element_type=jnp.float32)
                        + g_l1b, 0.0),
                    g_l2w, dn, preferred_element_type=jnp.float32) + g_l2b
                t = _ln(t + ff, g_ln2g, g_ln2b, 1e-5)
                logits = lax.dot_general(t, g_pw, dn,
                                         preferred_element_type=jnp.float32) + g_pb
                probs = jax.nn.softmax(logits, axis=-1)
                ent = -jnp.sum(probs * jnp.log(probs), axis=-1, keepdims=True)
                noisy = logits * ent + nr[bi * TOP_K + i]
                m1 = jnp.max(noisy, axis=-1, keepdims=True)
                i1 = jnp.min(jnp.where(noisy == m1, e_iota, NUM_EXPERTS),
                             axis=-1, keepdims=True)
                mask1 = e_iota == i1
                n2 = jnp.where(mask1, -jnp.inf, noisy)
                m2 = jnp.max(n2, axis=-1, keepdims=True)
                i2 = jnp.min(jnp.where(n2 == m2, e_iota, NUM_EXPERTS),
                             axis=-1, keepdims=True)
                sel = mask1 | (e_iota == i2)
                gw = jax.nn.softmax(
                    jnp.where(sel, noisy, jnp.float32(-1e9)), axis=-1)
                eout = jnp.zeros_like(boost)
                for e in range(NUM_EXPERTS):
                    w1, b1, w2, b2 = experts[e]
                    er = lax.dot_general(
                        jnp.maximum(
                            lax.dot_general(boost, w1, dn,
                                            preferred_element_type=jnp.float32)
                            + b1, 0.0),
                        w2, dn, preferred_element_type=jnp.float32) + b2
                    eout = eout + gw[..., e:e + 1] * er
                boost = boost + jnp.float32(ALPHA) * eout
            x = _ln(residual + boost, moe_g, moe_b, 1e-5)
            kl_gw = gw

        last_g, last_b = pop(), pop()
        feats = _ln(x, last_g, last_b, 1e-8)
        final = feats[:, L - 1:L, :]
        comb = final + ur[...][:, None, :]
        out_l[...] = jnp.sum(comb * ir[...][:, None, :], axis=-1)

        gws = jnp.sum(kl_gw, axis=(0, 1))[None, :]

        @pl.when(step == 0)
        def _():
            out_m[...] = jnp.zeros_like(out_m)

        acc = out_m[0:1, :] + gws
        out_m[0:1, :] = acc

        @pl.when(step == pl.num_programs(0) - 1)
        def _():
            avg = acc / jnp.float32(B * L)
            tgt = jnp.float32(1.0 / NUM_EXPERTS)
            kl = (jnp.sum(tgt * (jnp.log(tgt) - jnp.log(avg + 1e-8)))
                  / NUM_EXPERTS)
            out_m[1:2, :] = jnp.broadcast_to(kl, (1, NUM_EXPERTS))

    def cspec(a):
        return pl.BlockSpec(a.shape, lambda i, _n=a.ndim: (0,) * _n)

    in_specs = [
        pl.BlockSpec((BB, L, H), lambda i: (i, 0, 0)),
        pl.BlockSpec((BB, L), lambda i: (i, 0)),
        pl.BlockSpec((NUM_BLOCKS * TOP_K, BB, L, NUM_EXPERTS),
                     lambda i: (0, i, 0, 0)),
        pl.BlockSpec((BB, H), lambda i: (i, 0)),
        pl.BlockSpec((BB, H), lambda i: (i, 0)),
    ] + [cspec(a) for a in wlist]

    out_specs = [
        pl.BlockSpec((BB, 1), lambda i: (i, 0)),
        pl.BlockSpec((8, NUM_EXPERTS), lambda i: (0, 0)),
    ]
    out_shape = [
        jax.ShapeDtypeStruct((B, 1), jnp.float32),
        jax.ShapeDtypeStruct((8, NUM_EXPERTS), jnp.float32),
    ]
    return pl.pallas_call(
        body, grid=(grid,), in_specs=in_specs, out_specs=out_specs,
        out_shape=out_shape,
    )(x_rows, poss, noise, u_rows, i_rows, *wlist)


def _make_noise(B, L):
    ns = []
    base = jax.random.key(1)
    for bi in range(NUM_BLOCKS):
        rk = jax.random.fold_in(base, bi)
        for i in range(TOP_K):
            ns.append(jax.random.gumbel(jax.random.fold_in(rk, i),
                                        (B, L, NUM_EXPERTS)))
    return jnp.stack(ns)


def kernel(user_ids, log_seqs, item_ids, params):
    B, L = log_seqs.shape
    seq_idx = log_seqs.reshape(-1).astype(jnp.int32)
    seq_rows, u_rows, i_rows = _sc_gather(
        params["item_emb"], params["user_emb"], seq_idx,
        user_ids.astype(jnp.int32), item_ids.astype(jnp.int32))
    x_rows = seq_rows.reshape(B, L, HIDDEN)
    poss = (jnp.arange(1, L + 1, dtype=jnp.int32)[None, :]
            * (log_seqs != 0).astype(jnp.int32))
    noise = _make_noise(B, L)
    logits2, misc = _tc_forward(x_rows, poss, noise, u_rows, i_rows, params)
    return logits2[:, 0], misc[1, 0]


# speedup vs baseline: 1.4393x; 1.3118x over previous
import functools

import jax
import jax.numpy as jnp
from jax import lax
from jax.experimental import pallas as pl
from jax.experimental.pallas import tpu as pltpu
from jax.experimental.pallas import tpu_sc as plsc

HIDDEN = 64
NUM_HEADS = 2
NUM_BLOCKS = 2
NUM_EXPERTS = 4
TOP_K = 2
ALPHA = 0.5

BB = 8
NW = 32
SC_CHUNK = 800


def _sc_gather(item_emb, user_emb, seq_idx, user_ids, item_ids):
    n_seq = seq_idx.shape[0]
    batch = user_ids.shape[0]
    rows_per_w = n_seq // NW
    n_chunks = rows_per_w // SC_CHUNK
    b_per_w = batch // NW

    def body(item_hbm, user_hbm, sidx_hbm, uid_hbm, iid_hbm,
             seq_out, u_out, i_out, idx_v, rows_v, idx_s, rows_s, sem):
        wid = lax.axis_index("s") * 2 + lax.axis_index("c")
        base0 = wid * rows_per_w
        for c in range(n_chunks):
            b = base0 + c * SC_CHUNK
            pltpu.sync_copy(sidx_hbm.at[pl.ds(b, SC_CHUNK)], idx_v)
            pltpu.async_copy(item_hbm.at[idx_v], rows_v, sem).wait()
            pltpu.sync_copy(rows_v, seq_out.at[pl.ds(b, SC_CHUNK)])
        ub = wid * b_per_w
        pltpu.sync_copy(uid_hbm.at[pl.ds(ub, b_per_w)], idx_s)
        pltpu.async_copy(user_hbm.at[idx_s], rows_s, sem).wait()
        pltpu.sync_copy(rows_s, u_out.at[pl.ds(ub, b_per_w)])
        pltpu.sync_copy(iid_hbm.at[pl.ds(ub, b_per_w)], idx_s)
        pltpu.async_copy(item_hbm.at[idx_s], rows_s, sem).wait()
        pltpu.sync_copy(rows_s, i_out.at[pl.ds(ub, b_per_w)])

    kfn = pl.kernel(
        body,
        mesh=plsc.VectorSubcoreMesh(core_axis_name="c", subcore_axis_name="s"),
        out_type=[
            jax.ShapeDtypeStruct((n_seq, HIDDEN), jnp.float32),
            jax.ShapeDtypeStruct((batch, HIDDEN), jnp.float32),
            jax.ShapeDtypeStruct((batch, HIDDEN), jnp.float32),
        ],
        scratch_types=[
            pltpu.VMEM((SC_CHUNK,), jnp.int32),
            pltpu.VMEM((SC_CHUNK, HIDDEN), jnp.float32),
            pltpu.VMEM((b_per_w,), jnp.int32),
            pltpu.VMEM((b_per_w, HIDDEN), jnp.float32),
            pltpu.SemaphoreType.DMA,
        ],
        compiler_params=pltpu.CompilerParams(use_tc_tiling_on_sc=False),
    )
    return kfn(item_emb, user_emb, seq_idx, user_ids, item_ids)


def _flatten_params(params):
    out = []

    def add(a):
        out.append(a if a.ndim == 2 else a.reshape(1, -1))

    add(params["pos_emb"])
    for bp in params["blocks"]:
        add(bp["attn_ln_g"]); add(bp["attn_ln_b"])
        m = bp["mha"]
        add(m["qkv_w"][0:64].T)
        add(m["qkv_w"][64:192].T)
        add(m["qkv_b"]); add(m["out_w"].T); add(m["out_b"])
        add(bp["fwd_ln_g"]); add(bp["fwd_ln_b"])
        g = bp["moe"]["gate"]
        add(g["qkv_w"].T)
        add(g["qkv_b"]); add(g["out_w"].T); add(g["out_b"])
        add(g["lin1_w"].T); add(g["lin1_b"]); add(g["lin2_w"].T); add(g["lin2_b"])
        add(g["ln1_g"]); add(g["ln1_b"]); add(g["ln2_g"]); add(g["ln2_b"])
        add(g["proj_w"].T); add(g["proj_b"])
        eps_ = bp["moe"]["experts"]
        add(jnp.concatenate([ep["w1"].T for ep in eps_], axis=1))
        add(jnp.concatenate([ep["b1"] for ep in eps_])[None, :])
        add(jnp.concatenate([ep["w2"].T for ep in eps_], axis=0))
        add(jnp.stack([ep["b2"] for ep in eps_]))
        add(bp["moe"]["ln_g"]); add(bp["moe"]["ln_b"])
    add(params["last_ln_g"]); add(params["last_ln_b"])
    return out


def _ln(x, g, b, eps):
    mu = jnp.mean(x, axis=-1, keepdims=True)
    var = jnp.mean((x - mu) ** 2, axis=-1, keepdims=True)
    return (x - mu) / jnp.sqrt(var + eps) * g + b


_DN = (((2,), (0,)), ((), ()))


def _mm(x, w):
    return lax.dot_general(x, w, _DN, preferred_element_type=jnp.float32)


def _attn(q, k, v, out_w, out_b, cmask):
    dh = HIDDEN // NUM_HEADS
    outs = []
    for h in range(NUM_HEADS):
        sl = slice(h * dh, (h + 1) * dh)
        qh, kh, vh = q[..., sl], k[..., sl], v[..., sl]
        s = lax.dot_general(qh, kh, (((2,), (2,)), ((0,), (0,))),
                            preferred_element_type=jnp.float32)
        s = s / jnp.sqrt(jnp.float32(dh))
        if cmask is not None:
            s = jnp.where(cmask[None], jnp.float32(-1e9), s)
        a = jax.nn.softmax(s, axis=-1)
        outs.append(lax.dot_general(a, vh, (((2,), (1,)), ((0,), (0,))),
                                    preferred_element_type=jnp.float32))
    o = jnp.concatenate(outs, axis=-1)
    return _mm(o, out_w) + out_b


def _tc_forward(x_rows, poss, noise, u_rows, i_rows, params):
    B, L, H = x_rows.shape
    grid = B // BB
    wlist = _flatten_params(params)
    nw = len(wlist)

    def body(*refs):
        xr, pr, nr, ur, ir = refs[:5]
        w = [r[...] for r in refs[5:5 + nw]]
        out_l, out_m = refs[5 + nw], refs[6 + nw]
        pop = functools.partial(w.pop, 0)
        step = pl.program_id(0)

        pos_emb = pop()

        x = xr[...] * jnp.float32(8.0)
        pidx = pr[...]
        oh = (pidx[:, :, None]
              == lax.broadcasted_iota(jnp.int32, (1, 1, L + 1), 2)
              ).astype(jnp.float32)
        x = x + lax.dot_general(oh, pos_emb, (((2,), (0,)), ((), ())),
                                preferred_element_type=jnp.float32)

        row = lax.broadcasted_iota(jnp.int32, (L, L), 0)
        col = lax.broadcasted_iota(jnp.int32, (L, L), 1)
        cmask = col > row
        e_iota = lax.broadcasted_iota(jnp.int32, (BB, L, NUM_EXPERTS), 2)
        exp4 = (lax.broadcasted_iota(jnp.int32, (NUM_EXPERTS, 256), 1) // 64
                == lax.broadcasted_iota(jnp.int32, (NUM_EXPERTS, 256), 0)
                ).astype(jnp.float32)

        def expert_mix(h_in, gw_, w1c, b1c, w2s, b2m):
            hh = jnp.maximum(_mm(h_in, w1c) + b1c, 0.0)
            hh = hh * _mm(gw_, exp4)
            return _mm(hh, w2s) + _mm(gw_, b2m)

        kl_gw = None
        for bi in range(NUM_BLOCKS):
            attn_g, attn_b = pop(), pop()
            wq, wkv, qkv_b, ow, ob = pop(), pop(), pop(), pop(), pop()
            fwd_g, fwd_b = pop(), pop()
            g_qkv_w, g_qkv_b, g_ow, g_ob = pop(), pop(), pop(), pop()
            g_l1w, g_l1b, g_l2w, g_l2b = pop(), pop(), pop(), pop()
            g_ln1g, g_ln1b, g_ln2g, g_ln2b = pop(), pop(), pop(), pop()
            g_pw, g_pb = pop(), pop()
            w1c, b1c, w2s, b2m = pop(), pop(), pop(), pop()
            moe_g, moe_b = pop(), pop()

            Q = _ln(x, attn_g, attn_b, 1e-8)
            q = _mm(Q, wq) + qkv_b[:, 0:64]
            kv = _mm(x, wkv) + qkv_b[:, 64:192]
            x = Q + _attn(q, kv[..., 0:64], kv[..., 64:128], ow, ob, cmask)
            x = _ln(x, fwd_g, fwd_b, 1e-8)

            residual = x
            boost = x
            gw = None
            for i in range(TOP_K):
                qkv = _mm(boost, g_qkv_w) + g_qkv_b
                sa = _attn(qkv[..., 0:64], qkv[..., 64:128], qkv[..., 128:192],
                           g_ow, g_ob, None)
                t = _ln(boost + sa, g_ln1g, g_ln1b, 1e-5)
                ff = _mm(jnp.maximum(_mm(t, g_l1w) + g_l1b, 0.0), g_l2w) + g_l2b
                t = _ln(t + ff, g_ln2g, g_ln2b, 1e-5)
                logits = _mm(t, g_pw) + g_pb
                probs = jax.nn.softmax(logits, axis=-1)
                ent = -jnp.sum(probs * jnp.log(probs), axis=-1, keepdims=True)
                noisy = logits * ent + nr[bi * TOP_K + i]
                m1 = jnp.max(noisy, axis=-1, keepdims=True)
                i1 = jnp.min(jnp.where(noisy == m1, e_iota, NUM_EXPERTS),
                             axis=-1, keepdims=True)
                mask1 = e_iota == i1
                n2 = jnp.where(mask1, -jnp.inf, noisy)
                m2 = jnp.max(n2, axis=-1, keepdims=True)
                i2 = jnp.min(jnp.where(n2 == m2, e_iota, NUM_EXPERTS),
                             axis=-1, keepdims=True)
                sel = mask1 | (e_iota == i2)
                gw = jax.nn.softmax(
                    jnp.where(sel, noisy, jnp.float32(-1e9)), axis=-1)
                last_pass = bi == NUM_BLOCKS - 1 and i == TOP_K - 1
                if last_pass:
                    eout = expert_mix(boost[:, L - 1:L, :], gw[:, L - 1:L, :],
                                      w1c, b1c, w2s, b2m)
                    boost = boost[:, L - 1:L, :] + jnp.float32(ALPHA) * eout
                else:
                    eout = expert_mix(boost, gw, w1c, b1c, w2s, b2m)
                    boost = boost + jnp.float32(ALPHA) * eout
            if bi == NUM_BLOCKS - 1:
                x = _ln(residual[:, L - 1:L, :] + boost, moe_g, moe_b, 1e-5)
            else:
                x = _ln(residual + boost, moe_g, moe_b, 1e-5)
            kl_gw = gw

        last_g, last_b = pop(), pop()
        final = _ln(x, last_g, last_b, 1e-8)
        comb = final + ur[...][:, None, :]
        out_l[...] = jnp.sum(comb * ir[...][:, None, :], axis=-1)

        gws = jnp.sum(kl_gw, axis=(0, 1))[None, :]

        @pl.when(step == 0)
        def _():
            out_m[...] = jnp.zeros_like(out_m)

        acc = out_m[0:1, :] + gws
        out_m[0:1, :] = acc

        @pl.when(step == pl.num_programs(0) - 1)
        def _():
            avg = acc / jnp.float32(B * L)
            tgt = jnp.float32(1.0 / NUM_EXPERTS)
            kl = (jnp.sum(tgt * (jnp.log(tgt) - jnp.log(avg + 1e-8)))
                  / NUM_EXPERTS)
            out_m[1:2, :] = jnp.broadcast_to(kl, (1, NUM_EXPERTS))

    def cspec(a):
        return pl.BlockSpec(a.shape, lambda i, _n=a.ndim: (0,) * _n)

    in_specs = [
        pl.BlockSpec((BB, L, H), lambda i: (i, 0, 0)),
        pl.BlockSpec((BB, L), lambda i: (i, 0)),
        pl.BlockSpec((NUM_BLOCKS * TOP_K, BB, L, NUM_EXPERTS),
                     lambda i: (0, i, 0, 0)),
        pl.BlockSpec((BB, H), lambda i: (i, 0)),
        pl.BlockSpec((BB, H), lambda i: (i, 0)),
    ] + [cspec(a) for a in wlist]

    out_specs = [
        pl.BlockSpec((BB, 1), lambda i: (i, 0)),
        pl.BlockSpec((8, NUM_EXPERTS), lambda i: (0, 0)),
    ]
    out_shape = [
        jax.ShapeDtypeStruct((B, 1), jnp.float32),
        jax.ShapeDtypeStruct((8, NUM_EXPERTS), jnp.float32),
    ]
    return pl.pallas_call(
        body, grid=(grid,), in_specs=in_specs, out_specs=out_specs,
        out_shape=out_shape,
    )(x_rows, poss, noise, u_rows, i_rows, *wlist)


def _make_noise(B, L):
    ns = []
    base = jax.random.key(1)
    for bi in range(NUM_BLOCKS):
        rk = jax.random.fold_in(base, bi)
        for i in range(TOP_K):
            ns.append(jax.random.gumbel(jax.random.fold_in(rk, i),
                                        (B, L, NUM_EXPERTS)))
    return jnp.stack(ns)


def kernel(user_ids, log_seqs, item_ids, params):
    B, L = log_seqs.shape
    seq_idx = log_seqs.reshape(-1).astype(jnp.int32)
    seq_rows, u_rows, i_rows = _sc_gather(
        params["item_emb"], params["user_emb"], seq_idx,
        user_ids.astype(jnp.int32), item_ids.astype(jnp.int32))
    x_rows = seq_rows.reshape(B, L, HIDDEN)
    poss = (jnp.arange(1, L + 1, dtype=jnp.int32)[None, :]
            * (log_seqs != 0).astype(jnp.int32))
    noise = _make_noise(B, L)
    logits2, misc = _tc_forward(x_rows, poss, noise, u_rows, i_rows, params)
    return logits2[:, 0], misc[1, 0]
